# baseline (speedup 1.0000x reference)
import jax
import jax.numpy as jnp
from jax.experimental import pallas as pl

N_USER = 10000
N_POI = 10000
EMBED = 32
BATCH = 4096
BM = 400
BN = 512


def _prop_body(a_ref, x_ref, u_ref, pt_ref):
    i = pl.program_id(0)
    a = a_ref[...].astype(jnp.bfloat16)
    x = x_ref[...].astype(jnp.bfloat16)
    u = jnp.dot(a, x, preferred_element_type=jnp.float32)
    u_ref[...] = u
    pt = jax.lax.dot_general(u.astype(jnp.bfloat16), a,
                             (((0,), (0,)), ((), ())),
                             preferred_element_type=jnp.float32)

    @pl.when(i == 0)
    def _init():
        pt_ref[...] = pt

    @pl.when(i != 0)
    def _acc():
        pt_ref[...] = pt_ref[...] + pt


def _propagate(A, x):
    return pl.pallas_call(
        _prop_body,
        grid=(N_USER // BM,),
        in_specs=[
            pl.BlockSpec((BM, N_POI), lambda i: (i, 0)),
            pl.BlockSpec((N_POI, EMBED), lambda i: (0, 0)),
        ],
        out_specs=[
            pl.BlockSpec((BM, EMBED), lambda i: (i, 0)),
            pl.BlockSpec((EMBED, N_POI), lambda i: (0, 0)),
        ],
        out_shape=[
            jax.ShapeDtypeStruct((N_USER, EMBED), jnp.float32),
            jax.ShapeDtypeStruct((EMBED, N_POI), jnp.float32),
        ],
    )(A, x)


def _transpose_body(xt_ref, x_ref):
    x_ref[...] = xt_ref[...].T


def _transpose(xt):
    n = xt.shape[1]
    return pl.pallas_call(
        _transpose_body,
        out_shape=jax.ShapeDtypeStruct((n, EMBED), jnp.float32),
    )(xt)


def _combine_user_body(e_ref, l1_ref, l2_ref, w_ref, out_ref):
    def norm(x):
        return x * jax.lax.rsqrt(jnp.sum(x * x, axis=1, keepdims=True))

    s = e_ref[...] + norm(l1_ref[...]) + norm(l2_ref[...])
    out_ref[...] = jnp.dot(s, w_ref[...].T, preferred_element_type=jnp.float32)


def _combine_user(embed, l1, l2, w):
    n = embed.shape[0]
    return pl.pallas_call(
        _combine_user_body,
        out_shape=jax.ShapeDtypeStruct((n, EMBED), jnp.float32),
    )(embed, l1, l2, w)


def _combine_poi_body(e_ref, l1t_ref, l2t_ref, w_ref, out_ref):
    def norm_t(xt):
        return xt * jax.lax.rsqrt(jnp.sum(xt * xt, axis=0, keepdims=True))

    st = e_ref[...].T + norm_t(l1t_ref[...]) + norm_t(l2t_ref[...])
    out_ref[...] = jnp.dot(w_ref[...], st, preferred_element_type=jnp.float32)


def _combine_poi_t(embed, l1t, l2t, w):
    n = embed.shape[0]
    return pl.pallas_call(
        _combine_poi_body,
        out_shape=jax.ShapeDtypeStruct((EMBED, n), jnp.float32),
    )(embed, l1t, l2t, w)


def _upw_body(bf_ref, pft_ref, out_ref):
    out_ref[...] = jnp.dot(bf_ref[...], pft_ref[...],
                           preferred_element_type=jnp.float32)


def _up_weight(bf, pft):
    return pl.pallas_call(
        _upw_body,
        grid=(pl.cdiv(N_POI, BN),),
        in_specs=[
            pl.BlockSpec((BATCH, EMBED), lambda i: (0, 0)),
            pl.BlockSpec((EMBED, BN), lambda i: (0, i)),
        ],
        out_specs=pl.BlockSpec((BATCH, BN), lambda i: (0, i)),
        out_shape=jax.ShapeDtypeStruct((BATCH, N_POI), jnp.float32),
    )(bf, pft)


def kernel(up_behavior_graph, user_embed, poi_embed, batch_user, W_user, W_poi):
    A = up_behavior_graph
    u1, p1t = _propagate(A, poi_embed)
    p1 = _transpose(p1t)
    u2, p2t = _propagate(A, p1)
    user_feature = _combine_user(user_embed, u1, u2, W_user)
    poi_feature_t = _combine_poi_t(poi_embed, p1t, p2t, W_poi)
    batch_user_feature = jnp.take(user_feature, batch_user, axis=0)
    up_weight = _up_weight(batch_user_feature, poi_feature_t)
    return (up_weight, user_feature)


# speedup vs baseline: 1.0016x; 1.0016x over previous
import jax
import jax.numpy as jnp
from jax.experimental import pallas as pl

N_USER = 10000
N_POI = 10000
EMBED = 32
BATCH = 4096
BM = 200
BN = 512


def _prop_body(a_ref, x_ref, u_ref, pt_ref):
    i = pl.program_id(0)
    a = a_ref[...].astype(jnp.bfloat16)
    x = x_ref[...].astype(jnp.bfloat16)
    u = jnp.dot(a, x, preferred_element_type=jnp.float32)
    u_ref[...] = u
    pt = jax.lax.dot_general(u.astype(jnp.bfloat16), a,
                             (((0,), (0,)), ((), ())),
                             preferred_element_type=jnp.float32)

    @pl.when(i == 0)
    def _init():
        pt_ref[...] = pt

    @pl.when(i != 0)
    def _acc():
        pt_ref[...] = pt_ref[...] + pt


def _propagate(A, x):
    return pl.pallas_call(
        _prop_body,
        grid=(N_USER // BM,),
        in_specs=[
            pl.BlockSpec((BM, N_POI), lambda i: (i, 0)),
            pl.BlockSpec((N_POI, EMBED), lambda i: (0, 0)),
        ],
        out_specs=[
            pl.BlockSpec((BM, EMBED), lambda i: (i, 0)),
            pl.BlockSpec((EMBED, N_POI), lambda i: (0, 0)),
        ],
        out_shape=[
            jax.ShapeDtypeStruct((N_USER, EMBED), jnp.float32),
            jax.ShapeDtypeStruct((EMBED, N_POI), jnp.float32),
        ],
    )(A, x)


def _transpose_body(xt_ref, x_ref):
    x_ref[...] = xt_ref[...].T


def _transpose(xt):
    n = xt.shape[1]
    return pl.pallas_call(
        _transpose_body,
        out_shape=jax.ShapeDtypeStruct((n, EMBED), jnp.float32),
    )(xt)


def _combine_user_body(e_ref, l1_ref, l2_ref, w_ref, out_ref):
    def norm(x):
        return x * jax.lax.rsqrt(jnp.sum(x * x, axis=1, keepdims=True))

    s = e_ref[...] + norm(l1_ref[...]) + norm(l2_ref[...])
    out_ref[...] = jnp.dot(s, w_ref[...].T, preferred_element_type=jnp.float32)


def _combine_user(embed, l1, l2, w):
    n = embed.shape[0]
    return pl.pallas_call(
        _combine_user_body,
        out_shape=jax.ShapeDtypeStruct((n, EMBED), jnp.float32),
    )(embed, l1, l2, w)


def _combine_poi_body(e_ref, l1t_ref, l2t_ref, w_ref, out_ref):
    def norm_t(xt):
        return xt * jax.lax.rsqrt(jnp.sum(xt * xt, axis=0, keepdims=True))

    st = e_ref[...].T + norm_t(l1t_ref[...]) + norm_t(l2t_ref[...])
    out_ref[...] = jnp.dot(w_ref[...], st, preferred_element_type=jnp.float32)


def _combine_poi_t(embed, l1t, l2t, w):
    n = embed.shape[0]
    return pl.pallas_call(
        _combine_poi_body,
        out_shape=jax.ShapeDtypeStruct((EMBED, n), jnp.float32),
    )(embed, l1t, l2t, w)


def _upw_body(bf_ref, pft_ref, out_ref):
    out_ref[...] = jnp.dot(bf_ref[...], pft_ref[...],
                           preferred_element_type=jnp.float32)


def _up_weight(bf, pft):
    return pl.pallas_call(
        _upw_body,
        grid=(pl.cdiv(N_POI, BN),),
        in_specs=[
            pl.BlockSpec((BATCH, EMBED), lambda i: (0, 0)),
            pl.BlockSpec((EMBED, BN), lambda i: (0, i)),
        ],
        out_specs=pl.BlockSpec((BATCH, BN), lambda i: (0, i)),
        out_shape=jax.ShapeDtypeStruct((BATCH, N_POI), jnp.float32),
    )(bf, pft)


def kernel(up_behavior_graph, user_embed, poi_embed, batch_user, W_user, W_poi):
    A = up_behavior_graph
    u1, p1t = _propagate(A, poi_embed)
    p1 = _transpose(p1t)
    u2, p2t = _propagate(A, p1)
    user_feature = _combine_user(user_embed, u1, u2, W_user)
    poi_feature_t = _combine_poi_t(poi_embed, p1t, p2t, W_poi)
    batch_user_feature = jnp.take(user_feature, batch_user, axis=0)
    up_weight = _up_weight(batch_user_feature, poi_feature_t)
    return (up_weight, user_feature)


# speedup vs baseline: 1.0728x; 1.0711x over previous
import jax
import jax.numpy as jnp
from jax.experimental import pallas as pl

N_USER = 10000
N_POI = 10000
EMBED = 32
BATCH = 4096
BM = 400
BB = 512
PADW = 10112


def _prop_body(a_ref, x_ref, u_ref, pt_ref):
    i = pl.program_id(0)
    a = a_ref[...]
    u = jnp.dot(a, x_ref[...], preferred_element_type=jnp.float32)
    u_ref[...] = u
    pt = jax.lax.dot_general(u, a, (((0,), (0,)), ((), ())),
                             preferred_element_type=jnp.float32)

    @pl.when(i == 0)
    def _init():
        pt_ref[...] = pt

    @pl.when(i != 0)
    def _acc():
        pt_ref[...] = pt_ref[...] + pt


def _propagate(A, x):
    return pl.pallas_call(
        _prop_body,
        grid=(N_USER // BM,),
        in_specs=[
            pl.BlockSpec((BM, N_POI), lambda i: (i, 0)),
            pl.BlockSpec((N_POI, EMBED), lambda i: (0, 0)),
        ],
        out_specs=[
            pl.BlockSpec((BM, EMBED), lambda i: (i, 0)),
            pl.BlockSpec((EMBED, N_POI), lambda i: (0, 0)),
        ],
        out_shape=[
            jax.ShapeDtypeStruct((N_USER, EMBED), jnp.float32),
            jax.ShapeDtypeStruct((EMBED, N_POI), jnp.float32),
        ],
    )(A, x)


def _transpose_body(xt_ref, x_ref):
    x_ref[...] = xt_ref[...].T


def _transpose(xt):
    n = xt.shape[1]
    return pl.pallas_call(
        _transpose_body,
        out_shape=jax.ShapeDtypeStruct((n, EMBED), jnp.float32),
    )(xt)


def _combine_user_body(e_ref, l1_ref, l2_ref, w_ref, out_ref):
    def norm(x):
        return x * jax.lax.rsqrt(jnp.sum(x * x, axis=1, keepdims=True))

    s = e_ref[...] + norm(l1_ref[...]) + norm(l2_ref[...])
    out_ref[...] = jnp.dot(s, w_ref[...].T, preferred_element_type=jnp.float32)


def _combine_user(embed, l1, l2, w):
    n = embed.shape[0]
    return pl.pallas_call(
        _combine_user_body,
        out_shape=jax.ShapeDtypeStruct((n, EMBED), jnp.float32),
    )(embed, l1, l2, w)


def _combine_poi_body(e_ref, l1t_ref, l2t_ref, w_ref, out_ref):
    def norm_t(xt):
        return xt * jax.lax.rsqrt(jnp.sum(xt * xt, axis=0, keepdims=True))

    st = e_ref[...].T + norm_t(l1t_ref[...]) + norm_t(l2t_ref[...])
    out_ref[...] = jnp.dot(w_ref[...], st, preferred_element_type=jnp.float32)


def _combine_poi_t(embed, l1t, l2t, w):
    n = embed.shape[0]
    return pl.pallas_call(
        _combine_poi_body,
        out_shape=jax.ShapeDtypeStruct((EMBED, n), jnp.float32),
    )(embed, l1t, l2t, w)


def _upw_body(bf_ref, pft_ref, out_ref):
    r = jnp.dot(bf_ref[...], pft_ref[...], preferred_element_type=jnp.float32)
    out_ref[...] = jnp.pad(r, ((0, 0), (0, PADW - N_POI)))


def _up_weight_padded(bf, pft):
    return pl.pallas_call(
        _upw_body,
        grid=(BATCH // BB,),
        in_specs=[
            pl.BlockSpec((BB, EMBED), lambda i: (i, 0)),
            pl.BlockSpec((EMBED, N_POI), lambda i: (0, 0)),
        ],
        out_specs=pl.BlockSpec((BB, PADW), lambda i: (i, 0)),
        out_shape=jax.ShapeDtypeStruct((BATCH, PADW), jnp.float32),
    )(bf, pft)


def kernel(up_behavior_graph, user_embed, poi_embed, batch_user, W_user, W_poi):
    A = up_behavior_graph
    u1, p1t = _propagate(A, poi_embed)
    p1 = _transpose(p1t)
    u2, p2t = _propagate(A, p1)
    user_feature = _combine_user(user_embed, u1, u2, W_user)
    poi_feature_t = _combine_poi_t(poi_embed, p1t, p2t, W_poi)
    batch_user_feature = jnp.take(user_feature, batch_user, axis=0)
    up_weight = _up_weight_padded(batch_user_feature, poi_feature_t)[:, :N_POI]
    return (up_weight, user_feature)
